# speedup vs baseline: 1.3492x; 1.3492x over previous
import functools

import jax
import jax.numpy as jnp
from jax import lax
from jax.experimental import pallas as pl
from jax.experimental.pallas import tpu as pltpu
from jax.experimental.pallas import tpu_sc as plsc

_EPS = 1e-12
_NC = 2
_NS = 16
_CH = 64


def _sc_gather(weight, ids):
    (n,) = ids.shape
    _, d = weight.shape
    nw = _NC * _NS
    per_w = n // nw
    assert n % (8 * nw) == 0 and per_w % _CH == 0

    mesh = plsc.VectorSubcoreMesh(core_axis_name="c", subcore_axis_name="s")

    @functools.partial(
        pl.kernel,
        out_type=jax.ShapeDtypeStruct((n, d), jnp.float32),
        mesh=mesh,
        scratch_types=[
            pltpu.VMEM((per_w,), jnp.int32),
            pltpu.VMEM((_CH, d), jnp.float32),
            pltpu.SemaphoreType.DMA,
        ],
    )
    def k(table_hbm, idx_hbm, out_hbm, idx_v, rows_v, sem):
        wid = lax.axis_index("s") * _NC + lax.axis_index("c")
        base = wid * per_w
        pltpu.sync_copy(idx_hbm.at[pl.ds(base, per_w)], idx_v)

        @pl.loop(0, per_w, step=_CH)
        def _(c):
            pltpu.async_copy(
                table_hbm.at[idx_v.at[pl.ds(c, _CH)]], rows_v, sem
            ).wait()
            pltpu.sync_copy(rows_v, out_hbm.at[pl.ds(base + c, _CH)])

    return k(weight, ids)


def _ln_body(g_ref, p_ref, gamma_ref, beta_ref, o_ref):
    x = g_ref[...] + p_ref[...]
    mean = jnp.mean(x, axis=-1, keepdims=True)
    xc = x - mean
    var = jnp.mean(xc * xc, axis=-1, keepdims=True)
    normed = xc * lax.rsqrt(var + _EPS)
    o_ref[...] = normed * gamma_ref[...] + beta_ref[...]


def _tc_add_ln(g3, pos, gamma, beta, bs=256):
    b, s, d = g3.shape
    return pl.pallas_call(
        _ln_body,
        grid=(b, s // bs),
        in_specs=[
            pl.BlockSpec((1, bs, d), lambda i, j: (i, j, 0)),
            pl.BlockSpec((bs, d), lambda i, j: (j, 0)),
            pl.BlockSpec((d,), lambda i, j: (0,)),
            pl.BlockSpec((d,), lambda i, j: (0,)),
        ],
        out_specs=pl.BlockSpec((1, bs, d), lambda i, j: (i, j, 0)),
        out_shape=jax.ShapeDtypeStruct((b, s, d), jnp.float32),
    )(g3, pos, gamma, beta)


def kernel(input_ids, weight, position_embeddings, ln_gamma, ln_beta):
    b, s = input_ids.shape
    _, d = weight.shape
    ids = input_ids.reshape(-1).astype(jnp.int32)
    gathered = _sc_gather(weight, ids)
    g3 = gathered.reshape(b, s, d)
    return _tc_add_ln(g3, position_embeddings[:s], ln_gamma, ln_beta)


# speedup vs baseline: 1.3711x; 1.0162x over previous
import functools

import jax
import jax.numpy as jnp
from jax import lax
from jax.experimental import pallas as pl
from jax.experimental.pallas import tpu as pltpu
from jax.experimental.pallas import tpu_sc as plsc

_EPS = 1e-12
_NC = 2
_NS = 16
_CH = 32


def _sc_gather(weight, ids):
    (n,) = ids.shape
    _, d = weight.shape
    nw = _NC * _NS
    per_w = n // nw
    assert n % (8 * nw) == 0 and per_w % _CH == 0

    mesh = plsc.VectorSubcoreMesh(core_axis_name="c", subcore_axis_name="s")
    nch = per_w // _CH

    @functools.partial(
        pl.kernel,
        out_type=jax.ShapeDtypeStruct((n, d), jnp.float32),
        mesh=mesh,
        scratch_types=[
            pltpu.VMEM((per_w,), jnp.int32),
            pltpu.VMEM((_CH, d), jnp.float32),
            pltpu.VMEM((_CH, d), jnp.float32),
            pltpu.SemaphoreType.DMA,
            pltpu.SemaphoreType.DMA,
            pltpu.SemaphoreType.DMA,
            pltpu.SemaphoreType.DMA,
        ],
    )
    def k(table_hbm, idx_hbm, out_hbm, idx_v, rows_a, rows_b, gs_a, gs_b, ws_a, ws_b):
        wid = lax.axis_index("s") * _NC + lax.axis_index("c")
        base = wid * per_w
        pltpu.sync_copy(idx_hbm.at[pl.ds(base, per_w)], idx_v)

        bufs = [(rows_a, gs_a, ws_a), (rows_b, gs_b, ws_b)]
        gathers = [None] * nch
        writes = [None] * nch
        gathers[0] = pltpu.async_copy(
            table_hbm.at[idx_v.at[pl.ds(0, _CH)]], rows_a, gs_a
        )
        for j in range(nch):
            rows, _, ws = bufs[j % 2]
            if j + 1 < nch:
                rows_n, gs_n, _ = bufs[(j + 1) % 2]
                if j >= 1:
                    writes[j - 1].wait()
                gathers[j + 1] = pltpu.async_copy(
                    table_hbm.at[idx_v.at[pl.ds((j + 1) * _CH, _CH)]],
                    rows_n,
                    gs_n,
                )
            gathers[j].wait()
            writes[j] = pltpu.async_copy(
                rows, out_hbm.at[pl.ds(base + j * _CH, _CH)], ws
            )
        if nch >= 2:
            writes[nch - 2].wait()
        writes[nch - 1].wait()

    return k(weight, ids)


def _ln_body(g_ref, p_ref, gamma_ref, beta_ref, o_ref):
    x = g_ref[...] + p_ref[...]
    mean = jnp.mean(x, axis=-1, keepdims=True)
    xc = x - mean
    var = jnp.mean(xc * xc, axis=-1, keepdims=True)
    normed = xc * lax.rsqrt(var + _EPS)
    o_ref[...] = normed * gamma_ref[...] + beta_ref[...]


def _tc_add_ln(g3, pos, gamma, beta, bs=256):
    b, s, d = g3.shape
    return pl.pallas_call(
        _ln_body,
        grid=(s // bs, b),
        in_specs=[
            pl.BlockSpec((1, bs, d), lambda i, j: (j, i, 0)),
            pl.BlockSpec((bs, d), lambda i, j: (i, 0)),
            pl.BlockSpec((d,), lambda i, j: (0,)),
            pl.BlockSpec((d,), lambda i, j: (0,)),
        ],
        out_specs=pl.BlockSpec((1, bs, d), lambda i, j: (j, i, 0)),
        out_shape=jax.ShapeDtypeStruct((b, s, d), jnp.float32),
    )(g3, pos, gamma, beta)


def kernel(input_ids, weight, position_embeddings, ln_gamma, ln_beta):
    b, s = input_ids.shape
    _, d = weight.shape
    ids = input_ids.reshape(-1).astype(jnp.int32)
    gathered = _sc_gather(weight, ids)
    g3 = gathered.reshape(b, s, d)
    return _tc_add_ln(g3, position_embeddings[:s], ln_gamma, ln_beta)


# speedup vs baseline: 1.6342x; 1.1919x over previous
import functools

import jax
import jax.numpy as jnp
from jax import lax
from jax.experimental import pallas as pl
from jax.experimental.pallas import tpu as pltpu
from jax.experimental.pallas import tpu_sc as plsc

_EPS = 1e-12
_NC = 2
_NS = 16
_CH = 32


def _sc_gather(weight, ids):
    (n,) = ids.shape
    _, d = weight.shape
    nw = _NC * _NS
    per_w = n // nw
    assert n % (8 * nw) == 0 and per_w % _CH == 0

    mesh = plsc.VectorSubcoreMesh(core_axis_name="c", subcore_axis_name="s")
    nch = per_w // _CH

    @functools.partial(
        pl.kernel,
        out_type=jax.ShapeDtypeStruct((n, d), jnp.float32),
        mesh=mesh,
        scratch_types=[
            pltpu.VMEM((per_w,), jnp.int32),
            pltpu.VMEM((_CH, d), jnp.float32),
            pltpu.VMEM((_CH, d), jnp.float32),
            pltpu.SemaphoreType.DMA,
            pltpu.SemaphoreType.DMA,
            pltpu.SemaphoreType.DMA,
            pltpu.SemaphoreType.DMA,
        ],
    )
    def k(table_hbm, idx_hbm, out_hbm, idx_v, rows_a, rows_b, gs_a, gs_b, ws_a, ws_b):
        wid = lax.axis_index("s") * _NC + lax.axis_index("c")
        base = wid * per_w
        pltpu.sync_copy(idx_hbm.at[pl.ds(base, per_w)], idx_v)

        bufs = [(rows_a, gs_a, ws_a), (rows_b, gs_b, ws_b)]
        gathers = [None] * nch
        writes = [None] * nch
        gathers[0] = pltpu.async_copy(
            table_hbm.at[idx_v.at[pl.ds(0, _CH)]], rows_a, gs_a
        )
        for j in range(nch):
            rows, _, ws = bufs[j % 2]
            if j + 1 < nch:
                rows_n, gs_n, _ = bufs[(j + 1) % 2]
                if j >= 1:
                    writes[j - 1].wait()
                gathers[j + 1] = pltpu.async_copy(
                    table_hbm.at[idx_v.at[pl.ds((j + 1) * _CH, _CH)]],
                    rows_n,
                    gs_n,
                )
            gathers[j].wait()
            writes[j] = pltpu.async_copy(
                rows, out_hbm.at[pl.ds(base + j * _CH, _CH)], ws
            )
        if nch >= 2:
            writes[nch - 2].wait()
        writes[nch - 1].wait()

    return k(weight, ids)


def _ln_body(g_ref, p_ref, gamma_ref, beta_ref, o_ref):
    x = g_ref[...] + p_ref[...]
    mean = jnp.mean(x, axis=-1, keepdims=True)
    xc = x - mean
    var = jnp.mean(xc * xc, axis=-1, keepdims=True)
    normed = xc * lax.rsqrt(var + _EPS)
    o_ref[...] = normed * gamma_ref[...] + beta_ref[...]


def _ln_body2(g_ref, p_ref, gamma_ref, beta_ref, o_ref):
    x = g_ref[...] + p_ref[...]
    mean = jnp.mean(x, axis=-1, keepdims=True)
    xc = x - mean
    var = jnp.mean(xc * xc, axis=-1, keepdims=True)
    o_ref[...] = (xc * lax.rsqrt(var + _EPS)) * gamma_ref[...] + beta_ref[...]


def _tc_add_ln(g3, pos, gamma, beta, bs=256):
    b, s, d = g3.shape
    return pl.pallas_call(
        _ln_body,
        grid=(s // bs, b),
        in_specs=[
            pl.BlockSpec((1, bs, d), lambda i, j: (j, i, 0)),
            pl.BlockSpec((bs, d), lambda i, j: (i, 0)),
            pl.BlockSpec((d,), lambda i, j: (0,)),
            pl.BlockSpec((d,), lambda i, j: (0,)),
        ],
        out_specs=pl.BlockSpec((1, bs, d), lambda i, j: (j, i, 0)),
        out_shape=jax.ShapeDtypeStruct((b, s, d), jnp.float32),
    )(g3, pos, gamma, beta)


def _tc_add_ln_flat(g2, pos, gamma, beta, b, s, bs=2048):
    _, d = g2.shape
    spb = s // bs
    out = pl.pallas_call(
        _ln_body2,
        grid=(spb, b),
        in_specs=[
            pl.BlockSpec((bs, d), lambda i, j: (j * spb + i, 0)),
            pl.BlockSpec((bs, d), lambda i, j: (i, 0)),
            pl.BlockSpec((d,), lambda i, j: (0,)),
            pl.BlockSpec((d,), lambda i, j: (0,)),
        ],
        out_specs=pl.BlockSpec((bs, d), lambda i, j: (j * spb + i, 0)),
        out_shape=jax.ShapeDtypeStruct((b * s, d), jnp.float32),
    )(g2, pos, gamma, beta)
    return out.reshape(b, s, d)


def kernel(input_ids, weight, position_embeddings, ln_gamma, ln_beta):
    b, s = input_ids.shape
    _, d = weight.shape
    ids = input_ids.reshape(-1).astype(jnp.int32)
    gathered = _sc_gather(weight, ids)
    return _tc_add_ln_flat(gathered, position_embeddings[:s], ln_gamma, ln_beta, b, s)
